# baseline (speedup 1.0000x reference)
import functools

import jax
import jax.numpy as jnp
from jax import lax
from jax.experimental import pallas as pl
from jax.experimental.pallas import tpu as pltpu
from jax.experimental.pallas import tpu_sc as plsc

_K = 64
_D = 8192
_L = 16
_NVREG = _D // _L
_NC, _NS = 2, 16
_NW = _NC * _NS

_MSB = -0x80000000
_M31 = 0x7FFFFFFF


def _skey(v):
    bits = lax.bitcast_convert_type(v, jnp.int32)
    sgn = lax.shift_right_arithmetic(bits, 31)
    return bits ^ (sgn & jnp.int32(_M31))


def _digit(sk):
    return lax.shift_right_logical(sk ^ jnp.int32(_MSB), 24)


def _row_threshold(row_v, hist_v, ext_v):
    lane = lax.iota(jnp.int32, _L)
    zeros = jnp.zeros((_L,), jnp.int32)
    ones = jnp.ones((_L,), jnp.int32)

    @plsc.parallel_loop(0, 256, unroll=8)
    def _clr(i):
        hist_v[pl.ds(i * _L, _L)] = zeros

    @plsc.parallel_loop(0, _NVREG, unroll=8, carry=jnp.full((_L,), _MSB, jnp.int32))
    def vmax(i, vm):
        sk = _skey(row_v[pl.ds(i * _L, _L)])
        idx = (_digit(sk) << 4) | lane
        plsc.addupdate_scatter(hist_v, [idx], ones)
        return jnp.maximum(vm, sk)

    maxdig = _digit(jnp.max(vmax, axis=0))

    def bucket_count(d):
        return jnp.sum(hist_v[pl.ds(d * _L, _L)], axis=0)

    def wcond(c):
        _, above, cur = c
        return above + cur < _K

    def wstep(c):
        d, above, cur = c
        return (d - 1, above + cur, bucket_count(d - 1))

    bstar, above, _cur = lax.while_loop(
        wcond, wstep, (maxdig, jnp.int32(0), bucket_count(maxdig))
    )
    need = _K - above

    @plsc.parallel_loop(0, _NVREG, unroll=8, carry=zeros)
    def base(i, b):
        sk = _skey(row_v[pl.ds(i * _L, _L)])
        m = _digit(sk) == bstar
        plsc.store_scatter(ext_v, [(b << 4) | lane], sk, mask=m)
        return b + m.astype(jnp.int32)

    nv = jnp.max(base, axis=0)

    def bis(it, cand):
        test = cand | (jnp.int32(1) << (23 - it))
        stest = test ^ jnp.int32(_MSB)

        @plsc.parallel_loop(0, nv, carry=zeros)
        def acc(j, a):
            valid = base > j
            hit = ext_v[pl.ds(j * _L, _L)] >= stest
            return a + (hit & valid).astype(jnp.int32)

        cnt = jnp.sum(acc, axis=0)
        return jnp.where(cnt >= need, test, cand)

    cand = lax.fori_loop(0, 24, bis, lax.shift_left(bstar, 24))
    sthr = cand ^ jnp.int32(_MSB)
    thr_bits = jnp.where(sthr >= 0, sthr, sthr ^ jnp.int32(_M31))
    return lax.bitcast_convert_type(zeros + thr_bits, jnp.float32)


def _apply(row_v, out_v, thr):
    @plsc.parallel_loop(0, _NVREG, unroll=8)
    def _app(i):
        v = row_v[pl.ds(i * _L, _L)]
        out_v[pl.ds(i * _L, _L)] = jnp.where(v >= thr, v, jnp.float32(0.0))


def _sc_body(rpw, x_hbm, o_hbm, in0, in1, out0, out1, hist_v, ext_v, si0, si1, so0, so1):
    wid = lax.axis_index("s") * _NC + lax.axis_index("c")
    row0 = wid * rpw
    last = row0 + rpw - 1

    pltpu.async_copy(x_hbm.at[row0], in0, si0)
    pltpu.async_copy(x_hbm.at[row0 + 1], in1, si1)

    def half(t, r, in_v, out_v, sin, sout):
        pltpu.make_async_copy(x_hbm.at[0], in_v, sin).wait()
        thr = _row_threshold(in_v, hist_v, ext_v)
        _apply(in_v, out_v, thr)

        @pl.when(t > 0)
        def _():
            pltpu.make_async_copy(out_v, o_hbm.at[0], sout).wait()

        pltpu.async_copy(out_v, o_hbm.at[r], sout)
        pltpu.async_copy(x_hbm.at[jnp.minimum(r + 2, last)], in_v, sin)

    def pair(t, c):
        r = row0 + 2 * t
        half(t, r, in0, out0, si0, so0)
        half(t, r + 1, in1, out1, si1, so1)
        return c

    lax.fori_loop(0, rpw // 2, pair, 0)
    pltpu.make_async_copy(out0, o_hbm.at[0], so0).wait()
    pltpu.make_async_copy(out1, o_hbm.at[0], so1).wait()
    pltpu.make_async_copy(x_hbm.at[0], in0, si0).wait()
    pltpu.make_async_copy(x_hbm.at[0], in1, si1).wait()


@jax.jit
def kernel(features):
    B, L, D = features.shape
    n_rows = B * L
    rpw = n_rows // _NW
    x = features.reshape(n_rows, D)
    mesh = plsc.VectorSubcoreMesh(core_axis_name="c", subcore_axis_name="s")
    out = pl.kernel(
        functools.partial(_sc_body, rpw),
        out_type=jax.ShapeDtypeStruct((n_rows, D), jnp.float32),
        mesh=mesh,
        compiler_params=pltpu.CompilerParams(needs_layout_passes=False),
        scratch_types=[
            pltpu.VMEM((D,), jnp.float32),
            pltpu.VMEM((D,), jnp.float32),
            pltpu.VMEM((D,), jnp.float32),
            pltpu.VMEM((D,), jnp.float32),
            pltpu.VMEM((256 * _L,), jnp.int32),
            pltpu.VMEM((D + _L,), jnp.int32),
            pltpu.SemaphoreType.DMA,
            pltpu.SemaphoreType.DMA,
            pltpu.SemaphoreType.DMA,
            pltpu.SemaphoreType.DMA,
        ],
    )(x)
    return out.reshape(B, L, D)


# speedup vs baseline: 1.4002x; 1.4002x over previous
import functools

import jax
import jax.numpy as jnp
from jax import lax
from jax.experimental import pallas as pl
from jax.experimental.pallas import tpu as pltpu
from jax.experimental.pallas import tpu_sc as plsc

_K = 64
_D = 8192
_L = 16
_NVREG = _D // _L
_NC, _NS = 2, 16
_NW = _NC * _NS

_MSB = -0x80000000
_M31 = 0x7FFFFFFF


def _skey(v):
    bits = lax.bitcast_convert_type(v, jnp.int32)
    sgn = lax.shift_right_arithmetic(bits, 31)
    return bits ^ (sgn & jnp.int32(_M31))


def _digit(sk):
    return lax.shift_right_logical(sk ^ jnp.int32(_MSB), 24)


def _row_threshold(row_v, hist_v, ext_v):
    lane = lax.iota(jnp.int32, _L)
    zeros = jnp.zeros((_L,), jnp.int32)
    ones = jnp.ones((_L,), jnp.int32)

    @plsc.parallel_loop(0, 256, unroll=8)
    def _clr(i):
        hist_v[pl.ds(i * _L, _L)] = zeros

    @plsc.parallel_loop(0, _NVREG, unroll=8, carry=jnp.full((_L,), _MSB, jnp.int32))
    def vmax(i, vm):
        sk = _skey(row_v[pl.ds(i * _L, _L)])
        idx = (_digit(sk) << 4) | lane
        plsc.addupdate_scatter(hist_v, [idx], ones)
        return jnp.maximum(vm, sk)

    maxdig = _digit(jnp.max(vmax, axis=0))

    def bucket_count(d):
        return jnp.sum(hist_v[pl.ds(d * _L, _L)], axis=0)

    def wcond(c):
        _, above, cur = c
        return above + cur < _K

    def wstep(c):
        d, above, cur = c
        return (d - 1, above + cur, bucket_count(d - 1))

    bstar, above, _cur = lax.while_loop(
        wcond, wstep, (maxdig, jnp.int32(0), bucket_count(maxdig))
    )
    need = _K - above

    @plsc.parallel_loop(0, _NVREG, unroll=8, carry=zeros)
    def base(i, b):
        sk = _skey(row_v[pl.ds(i * _L, _L)])
        m = _digit(sk) == bstar
        plsc.store_scatter(ext_v, [(b << 4) | lane], sk, mask=m)
        return b + m.astype(jnp.int32)

    nv = jnp.max(base, axis=0)

    def bcond(c):
        it, _, done = c
        return (it < 24) & jnp.logical_not(done)

    def bbody(c):
        it, cand, _ = c
        test = cand | (jnp.int32(1) << (23 - it))
        stest = test ^ jnp.int32(_MSB)

        @plsc.parallel_loop(0, nv, carry=zeros)
        def acc(j, a):
            valid = base > j
            hit = ext_v[pl.ds(j * _L, _L)] >= stest
            return a + (hit & valid).astype(jnp.int32)

        cnt = jnp.sum(acc, axis=0)
        return (it + 1, jnp.where(cnt >= need, test, cand), cnt == need)

    _, cand, _ = lax.while_loop(
        bcond, bbody, (jnp.int32(0), lax.shift_left(bstar, 24), need < 0)
    )
    sthr = cand ^ jnp.int32(_MSB)
    thr_bits = jnp.where(sthr >= 0, sthr, sthr ^ jnp.int32(_M31))
    return lax.bitcast_convert_type(zeros + thr_bits, jnp.float32)


def _apply(row_v, out_v, thr):
    @plsc.parallel_loop(0, _NVREG, unroll=8)
    def _app(i):
        v = row_v[pl.ds(i * _L, _L)]
        out_v[pl.ds(i * _L, _L)] = jnp.where(v >= thr, v, jnp.float32(0.0))


def _sc_body(rpw, x_hbm, o_hbm, in0, in1, out0, out1, hist_v, ext_v, si0, si1, so0, so1):
    wid = lax.axis_index("s") * _NC + lax.axis_index("c")
    row0 = wid * rpw
    last = row0 + rpw - 1

    pltpu.async_copy(x_hbm.at[row0], in0, si0)
    pltpu.async_copy(x_hbm.at[row0 + 1], in1, si1)

    def half(t, r, in_v, out_v, sin, sout):
        pltpu.make_async_copy(x_hbm.at[0], in_v, sin).wait()
        thr = _row_threshold(in_v, hist_v, ext_v)
        _apply(in_v, out_v, thr)

        @pl.when(t > 0)
        def _():
            pltpu.make_async_copy(out_v, o_hbm.at[0], sout).wait()

        pltpu.async_copy(out_v, o_hbm.at[r], sout)
        pltpu.async_copy(x_hbm.at[jnp.minimum(r + 2, last)], in_v, sin)

    def pair(t, c):
        r = row0 + 2 * t
        half(t, r, in0, out0, si0, so0)
        half(t, r + 1, in1, out1, si1, so1)
        return c

    lax.fori_loop(0, rpw // 2, pair, 0)
    pltpu.make_async_copy(out0, o_hbm.at[0], so0).wait()
    pltpu.make_async_copy(out1, o_hbm.at[0], so1).wait()
    pltpu.make_async_copy(x_hbm.at[0], in0, si0).wait()
    pltpu.make_async_copy(x_hbm.at[0], in1, si1).wait()


@jax.jit
def kernel(features):
    B, L, D = features.shape
    n_rows = B * L
    rpw = n_rows // _NW
    x = features.reshape(n_rows, D)
    mesh = plsc.VectorSubcoreMesh(core_axis_name="c", subcore_axis_name="s")
    out = pl.kernel(
        functools.partial(_sc_body, rpw),
        out_type=jax.ShapeDtypeStruct((n_rows, D), jnp.float32),
        mesh=mesh,
        compiler_params=pltpu.CompilerParams(needs_layout_passes=False),
        scratch_types=[
            pltpu.VMEM((D,), jnp.float32),
            pltpu.VMEM((D,), jnp.float32),
            pltpu.VMEM((D,), jnp.float32),
            pltpu.VMEM((D,), jnp.float32),
            pltpu.VMEM((256 * _L,), jnp.int32),
            pltpu.VMEM((D + _L,), jnp.int32),
            pltpu.SemaphoreType.DMA,
            pltpu.SemaphoreType.DMA,
            pltpu.SemaphoreType.DMA,
            pltpu.SemaphoreType.DMA,
        ],
    )(x)
    return out.reshape(B, L, D)


# speedup vs baseline: 1.4446x; 1.0317x over previous
import functools

import jax
import jax.numpy as jnp
from jax import lax
from jax.experimental import pallas as pl
from jax.experimental.pallas import tpu as pltpu
from jax.experimental.pallas import tpu_sc as plsc

_K = 64
_D = 8192
_L = 16
_NVREG = _D // _L
_NC, _NS = 2, 16
_NW = _NC * _NS

_MSB = -0x80000000
_M31 = 0x7FFFFFFF


def _skey(v):
    bits = lax.bitcast_convert_type(v, jnp.int32)
    sgn = lax.shift_right_arithmetic(bits, 31)
    return bits ^ (sgn & jnp.int32(_M31))


def _digit(sk):
    return lax.shift_right_logical(sk ^ jnp.int32(_MSB), 24)


def _row_threshold(row_v, hist_v, ext_v):
    lane = lax.iota(jnp.int32, _L)
    zeros = jnp.zeros((_L,), jnp.int32)
    ones = jnp.ones((_L,), jnp.int32)

    @plsc.parallel_loop(0, 256, unroll=8)
    def _clr(i):
        hist_v[pl.ds(i * _L, _L)] = zeros

    @plsc.parallel_loop(0, _NVREG, unroll=8, carry=jnp.full((_L,), _MSB, jnp.int32))
    def vmax(i, vm):
        sk = _skey(row_v[pl.ds(i * _L, _L)])
        idx = (_digit(sk) << 4) | lane
        plsc.addupdate_scatter(hist_v, [idx], ones)
        return jnp.maximum(vm, sk)

    maxdig = _digit(jnp.max(vmax, axis=0))

    def bucket_count(d):
        return jnp.sum(hist_v[pl.ds(d * _L, _L)], axis=0)

    def wcond(c):
        _, above, cur = c
        return above + cur < _K

    def wstep(c):
        d, above, cur = c
        return (d - 1, above + cur, bucket_count(d - 1))

    bstar, above, _cur = lax.while_loop(
        wcond, wstep, (maxdig, jnp.int32(0), bucket_count(maxdig))
    )
    need = _K - above

    @plsc.parallel_loop(0, _NVREG, unroll=8, carry=zeros)
    def base(i, b):
        sk = _skey(row_v[pl.ds(i * _L, _L)])
        m = _digit(sk) == bstar
        plsc.store_scatter(ext_v, [(b << 4) | lane], sk, mask=m)
        return b + m.astype(jnp.int32)

    nv = jnp.max(base, axis=0)

    def bcond(c):
        it, _, done = c
        return (it < 24) & jnp.logical_not(done)

    def bbody(c):
        it, cand, _ = c
        test = cand | (jnp.int32(1) << (23 - it))
        stest = test ^ jnp.int32(_MSB)

        @plsc.parallel_loop(0, nv, carry=zeros)
        def acc(j, a):
            valid = base > j
            hit = ext_v[pl.ds(j * _L, _L)] >= stest
            return a + (hit & valid).astype(jnp.int32)

        cnt = jnp.sum(acc, axis=0)
        return (it + 1, jnp.where(cnt >= need, test, cand), cnt == need)

    _, cand, _ = lax.while_loop(
        bcond, bbody, (jnp.int32(0), lax.shift_left(bstar, 24), need < 0)
    )
    sthr = cand ^ jnp.int32(_MSB)
    thr_bits = jnp.where(sthr >= 0, sthr, sthr ^ jnp.int32(_M31))
    return lax.bitcast_convert_type(zeros + thr_bits, jnp.float32)


def _apply(row_v, out_v, thr):
    @plsc.parallel_loop(0, _NVREG, unroll=8)
    def _app(i):
        v = row_v[pl.ds(i * _L, _L)]
        out_v[pl.ds(i * _L, _L)] = jnp.where(v >= thr, v, jnp.float32(0.0))


def _sc_body(rpw, x_hbm, o_hbm, in0, in1, out0, out1, hist_v, ext_v, si0, si1, so0, so1):
    wid = lax.axis_index("s") * _NC + lax.axis_index("c")
    row0 = wid * rpw
    last = row0 + rpw - 1

    pltpu.async_copy(x_hbm.at[row0], in0, si0)
    pltpu.async_copy(x_hbm.at[row0 + 1], in1, si1)

    def half(t, r, in_v, out_v, sin, sout):
        pltpu.make_async_copy(x_hbm.at[0], in_v, sin).wait()
        thr = _row_threshold(in_v, hist_v, ext_v)
        _apply(in_v, out_v, thr)

        @pl.when(t > 0)
        def _():
            pltpu.make_async_copy(out_v, o_hbm.at[0], sout).wait()

        pltpu.async_copy(out_v, o_hbm.at[r], sout)
        pltpu.async_copy(x_hbm.at[jnp.minimum(r + 2, last)], in_v, sin)

    def pair(t, c):
        r = row0 + 2 * t
        half(t, r, in0, out0, si0, so0)
        half(t, r + 1, in1, out1, si1, so1)
        return c

    lax.fori_loop(0, rpw // 2, pair, 0)
    pltpu.make_async_copy(out0, o_hbm.at[0], so0).wait()
    pltpu.make_async_copy(out1, o_hbm.at[0], so1).wait()
    pltpu.make_async_copy(x_hbm.at[0], in0, si0).wait()
    pltpu.make_async_copy(x_hbm.at[0], in1, si1).wait()


def _sc_topk(x):
    n_rows, D = x.shape
    rpw = n_rows // _NW
    mesh = plsc.VectorSubcoreMesh(core_axis_name="c", subcore_axis_name="s")
    return pl.kernel(
        functools.partial(_sc_body, rpw),
        out_type=jax.ShapeDtypeStruct((n_rows, D), jnp.float32),
        mesh=mesh,
        compiler_params=pltpu.CompilerParams(needs_layout_passes=False),
        scratch_types=[
            pltpu.VMEM((D,), jnp.float32),
            pltpu.VMEM((D,), jnp.float32),
            pltpu.VMEM((D,), jnp.float32),
            pltpu.VMEM((D,), jnp.float32),
            pltpu.VMEM((256 * _L,), jnp.int32),
            pltpu.VMEM((D + _L,), jnp.int32),
            pltpu.SemaphoreType.DMA,
            pltpu.SemaphoreType.DMA,
            pltpu.SemaphoreType.DMA,
            pltpu.SemaphoreType.DMA,
        ],
    )(x)


def _tc_block(x_ref, o_ref):
    x = x_ref[...]
    bits = lax.bitcast_convert_type(x, jnp.int32)
    sgn = lax.shift_right_arithmetic(bits, 31)
    skey = bits ^ (sgn & jnp.int32(_M31))

    def body(i, ku):
        cand_u = ku | (jnp.int32(1) << (31 - i))
        scand = cand_u ^ jnp.int32(_MSB)
        cnt = jnp.sum((skey >= scand).astype(jnp.float32), axis=1, keepdims=True)
        return jnp.where(cnt >= _K, cand_u, ku)

    ku = lax.fori_loop(0, 32, body, jnp.zeros((x.shape[0], 1), jnp.int32), unroll=True)
    o_ref[...] = jnp.where(skey >= (ku ^ jnp.int32(_MSB)), x, jnp.float32(0.0))


def _tc_topk(x):
    n_rows, D = x.shape
    r = 128
    return pl.pallas_call(
        _tc_block,
        grid=(n_rows // r,),
        in_specs=[pl.BlockSpec((r, D), lambda i: (i, 0))],
        out_specs=pl.BlockSpec((r, D), lambda i: (i, 0)),
        out_shape=jax.ShapeDtypeStruct((n_rows, D), jnp.float32),
    )(x)


_TC_ROWS = 1920


@jax.jit
def kernel(features):
    B, L, D = features.shape
    n_rows = B * L
    x = features.reshape(n_rows, D)
    out_sc = _sc_topk(x[_TC_ROWS:])
    out_tc = _tc_topk(x[:_TC_ROWS])
    return jnp.concatenate([out_tc, out_sc], axis=0).reshape(B, L, D)


# speedup vs baseline: 1.4460x; 1.0009x over previous
import functools

import jax
import jax.numpy as jnp
from jax import lax
from jax.experimental import pallas as pl
from jax.experimental.pallas import tpu as pltpu
from jax.experimental.pallas import tpu_sc as plsc

_K = 64
_D = 8192
_L = 16
_NVREG = _D // _L
_NC, _NS = 2, 16
_NW = _NC * _NS

_MSB = -0x80000000
_M31 = 0x7FFFFFFF


def _skey(v):
    bits = lax.bitcast_convert_type(v, jnp.int32)
    sgn = lax.shift_right_arithmetic(bits, 31)
    return bits ^ (sgn & jnp.int32(_M31))


def _digit(sk):
    return lax.shift_right_logical(sk ^ jnp.int32(_MSB), 24)


def _row_threshold(row_v, hist_v, ext_v):
    lane = lax.iota(jnp.int32, _L)
    zeros = jnp.zeros((_L,), jnp.int32)
    ones = jnp.ones((_L,), jnp.int32)

    @plsc.parallel_loop(0, 256, unroll=8)
    def _clr(i):
        hist_v[pl.ds(i * _L, _L)] = zeros

    @plsc.parallel_loop(0, _NVREG, unroll=8, carry=jnp.full((_L,), _MSB, jnp.int32))
    def vmax(i, vm):
        sk = _skey(row_v[pl.ds(i * _L, _L)])
        idx = (_digit(sk) << 4) | lane
        plsc.addupdate_scatter(hist_v, [idx], ones)
        return jnp.maximum(vm, sk)

    maxdig = _digit(jnp.max(vmax, axis=0))

    def bucket_count(d):
        return jnp.sum(hist_v[pl.ds(d * _L, _L)], axis=0)

    def wcond(c):
        _, above, cur = c
        return above + cur < _K

    def wstep(c):
        d, above, cur = c
        return (d - 1, above + cur, bucket_count(d - 1))

    bstar, above, _cur = lax.while_loop(
        wcond, wstep, (maxdig, jnp.int32(0), bucket_count(maxdig))
    )
    need = _K - above

    @plsc.parallel_loop(0, _NVREG, unroll=8, carry=zeros)
    def base(i, b):
        sk = _skey(row_v[pl.ds(i * _L, _L)])
        m = _digit(sk) == bstar
        plsc.store_scatter(ext_v, [(b << 4) | lane], sk, mask=m)
        return b + m.astype(jnp.int32)

    nv = jnp.max(base, axis=0)

    def bcond(c):
        it, _, done = c
        return (it < 24) & jnp.logical_not(done)

    def bbody(c):
        it, cand, _ = c
        test = cand | (jnp.int32(1) << (23 - it))
        stest = test ^ jnp.int32(_MSB)

        @plsc.parallel_loop(0, nv, carry=zeros)
        def acc(j, a):
            valid = base > j
            hit = ext_v[pl.ds(j * _L, _L)] >= stest
            return a + (hit & valid).astype(jnp.int32)

        cnt = jnp.sum(acc, axis=0)
        return (it + 1, jnp.where(cnt >= need, test, cand), cnt == need)

    _, cand, _ = lax.while_loop(
        bcond, bbody, (jnp.int32(0), lax.shift_left(bstar, 24), need < 0)
    )
    sthr = cand ^ jnp.int32(_MSB)
    thr_bits = jnp.where(sthr >= 0, sthr, sthr ^ jnp.int32(_M31))
    return lax.bitcast_convert_type(zeros + thr_bits, jnp.float32)


def _apply(row_v, out_v, thr):
    @plsc.parallel_loop(0, _NVREG, unroll=8)
    def _app(i):
        v = row_v[pl.ds(i * _L, _L)]
        out_v[pl.ds(i * _L, _L)] = jnp.where(v >= thr, v, jnp.float32(0.0))


def _sc_body(rpw, x_hbm, o_hbm, in0, in1, out0, out1, hist_v, ext_v, si0, si1, so0, so1):
    wid = lax.axis_index("s") * _NC + lax.axis_index("c")
    row0 = wid * rpw
    last = row0 + rpw - 1

    pltpu.async_copy(x_hbm.at[row0], in0, si0)
    pltpu.async_copy(x_hbm.at[row0 + 1], in1, si1)

    def half(t, r, in_v, out_v, sin, sout):
        pltpu.make_async_copy(x_hbm.at[0], in_v, sin).wait()
        thr = _row_threshold(in_v, hist_v, ext_v)
        _apply(in_v, out_v, thr)

        @pl.when(t > 0)
        def _():
            pltpu.make_async_copy(out_v, o_hbm.at[0], sout).wait()

        pltpu.async_copy(out_v, o_hbm.at[r], sout)
        pltpu.async_copy(x_hbm.at[jnp.minimum(r + 2, last)], in_v, sin)

    def pair(t, c):
        r = row0 + 2 * t
        half(t, r, in0, out0, si0, so0)
        half(t, r + 1, in1, out1, si1, so1)
        return c

    lax.fori_loop(0, rpw // 2, pair, 0)
    pltpu.make_async_copy(out0, o_hbm.at[0], so0).wait()
    pltpu.make_async_copy(out1, o_hbm.at[0], so1).wait()
    pltpu.make_async_copy(x_hbm.at[0], in0, si0).wait()
    pltpu.make_async_copy(x_hbm.at[0], in1, si1).wait()


def _sc_topk(x):
    n_rows, D = x.shape
    rpw = n_rows // _NW
    mesh = plsc.VectorSubcoreMesh(core_axis_name="c", subcore_axis_name="s")
    return pl.kernel(
        functools.partial(_sc_body, rpw),
        out_type=jax.ShapeDtypeStruct((n_rows, D), jnp.float32),
        mesh=mesh,
        compiler_params=pltpu.CompilerParams(needs_layout_passes=False),
        cost_estimate=pl.CostEstimate(
            flops=30 * n_rows * D, transcendentals=0, bytes_accessed=8 * n_rows * D
        ),
        scratch_types=[
            pltpu.VMEM((D,), jnp.float32),
            pltpu.VMEM((D,), jnp.float32),
            pltpu.VMEM((D,), jnp.float32),
            pltpu.VMEM((D,), jnp.float32),
            pltpu.VMEM((256 * _L,), jnp.int32),
            pltpu.VMEM((D + _L,), jnp.int32),
            pltpu.SemaphoreType.DMA,
            pltpu.SemaphoreType.DMA,
            pltpu.SemaphoreType.DMA,
            pltpu.SemaphoreType.DMA,
        ],
    )(x)


def _tc_block(x_ref, o_ref):
    x = x_ref[...]
    bits = lax.bitcast_convert_type(x, jnp.int32)
    sgn = lax.shift_right_arithmetic(bits, 31)
    skey = bits ^ (sgn & jnp.int32(_M31))

    def body(i, ku):
        cand_u = ku | (jnp.int32(1) << (31 - i))
        scand = cand_u ^ jnp.int32(_MSB)
        cnt = jnp.sum((skey >= scand).astype(jnp.float32), axis=1, keepdims=True)
        return jnp.where(cnt >= _K, cand_u, ku)

    ku = lax.fori_loop(0, 32, body, jnp.zeros((x.shape[0], 1), jnp.int32), unroll=True)
    o_ref[...] = jnp.where(skey >= (ku ^ jnp.int32(_MSB)), x, jnp.float32(0.0))


def _tc_topk(x):
    n_rows, D = x.shape
    r = 128
    return pl.pallas_call(
        _tc_block,
        grid=(n_rows // r,),
        in_specs=[pl.BlockSpec((r, D), lambda i: (i, 0))],
        out_specs=pl.BlockSpec((r, D), lambda i: (i, 0)),
        out_shape=jax.ShapeDtypeStruct((n_rows, D), jnp.float32),
    )(x)


_TC_ROWS = 1920


@jax.jit
def kernel(features):
    B, L, D = features.shape
    n_rows = B * L
    x = features.reshape(n_rows, D)
    out_sc = _sc_topk(x[_TC_ROWS:])
    out_tc = _tc_topk(x[:_TC_ROWS])
    return jnp.concatenate([out_tc, out_sc], axis=0).reshape(B, L, D)
